# Initial kernel scaffold; baseline (speedup 1.0000x reference)
import jax
import jax.numpy as jnp
from jax.experimental import pallas as pl


def kernel(non_scale_dense, sparse_depth):
    raise NotImplementedError("write your pallas kernel here")


# speedup vs baseline: 1.0564x; 1.0564x over previous
import jax
import jax.numpy as jnp
from jax.experimental import pallas as pl

MAX_DEPTH = 20.0
VALID_THRESHOLD = 1e-06
MIN_VALID_POINTS = 10


def _body(x_ref, y_ref, o_ref):
    x = x_ref[0]
    y = y_ref[0]
    m = ((y > VALID_THRESHOLD) & (y <= MAX_DEPTH)).astype(x.dtype)
    xm = x * m
    ym = y * m
    n = jnp.sum(m)
    x_sum = jnp.sum(xm)
    x_sq_sum = jnp.sum(x * xm)
    y_sum = jnp.sum(ym)
    xy_sum = jnp.sum(x * ym)
    det = n * x_sq_sum - x_sum * x_sum
    valid = (n >= MIN_VALID_POINTS) & (jnp.abs(det) >= 1e-08)
    safe_det = jnp.where(valid, det, 1.0)
    scale = jnp.where(valid, (n * xy_sum - x_sum * y_sum) / safe_det, 1.0)
    bias = jnp.where(valid, (x_sq_sum * y_sum - x_sum * xy_sum) / safe_det, 0.0)
    o_ref[0] = scale * x + bias


def kernel(non_scale_dense, sparse_depth):
    b, c, h, w = non_scale_dense.shape
    x = non_scale_dense.reshape(b, h, w)
    y = sparse_depth.reshape(b, h, w)
    out = pl.pallas_call(
        _body,
        grid=(b,),
        in_specs=[
            pl.BlockSpec((1, h, w), lambda i: (i, 0, 0)),
            pl.BlockSpec((1, h, w), lambda i: (i, 0, 0)),
        ],
        out_specs=pl.BlockSpec((1, h, w), lambda i: (i, 0, 0)),
        out_shape=jax.ShapeDtypeStruct((b, h, w), x.dtype),
    )(x, y)
    return out.reshape(b, c, h, w)
